# baseline (speedup 1.0000x reference)
import functools

import jax
import jax.numpy as jnp
from jax import lax
from jax.experimental import pallas as pl
from jax.experimental.pallas import tpu as pltpu
from jax.experimental.pallas import tpu_sc as plsc

BATCH = 16384
D_FEAT = 128
EMBED_SIZE = 64

NC = 2
NS = 16
NW = NC * NS
B_PER_W = BATCH // NW
IDX_CHUNK = 128
N_CHUNK = B_PER_W // IDX_CHUNK


def _sc_gather_body(ids_f_hbm, ids_e_hbm, feat_hbm, emb_hbm,
                    out_feat_hbm, out_emb_hbm,
                    idx_f, idx_e, rows_f, rows_e, sem_f, sem_e, sem_out):
    wid = lax.axis_index("s") * NC + lax.axis_index("c")
    base = wid * B_PER_W
    pltpu.sync_copy(ids_f_hbm.at[wid], idx_f)
    pltpu.sync_copy(ids_e_hbm.at[wid], idx_e)
    for j in range(N_CHUNK):
        pltpu.async_copy(feat_hbm.at[idx_f.at[j]],
                         rows_f.at[pl.ds(j * IDX_CHUNK, IDX_CHUNK)], sem_f)
    for j in range(N_CHUNK):
        pltpu.async_copy(emb_hbm.at[idx_e.at[j]],
                         rows_e.at[pl.ds(j * IDX_CHUNK, IDX_CHUNK)], sem_e)
    for j in range(N_CHUNK):
        pltpu.make_async_copy(feat_hbm.at[idx_f.at[j]],
                              rows_f.at[pl.ds(j * IDX_CHUNK, IDX_CHUNK)],
                              sem_f).wait()
    cp_f = pltpu.async_copy(rows_f, out_feat_hbm.at[pl.ds(base, B_PER_W)],
                            sem_out)
    for j in range(N_CHUNK):
        pltpu.make_async_copy(emb_hbm.at[idx_e.at[j]],
                              rows_e.at[pl.ds(j * IDX_CHUNK, IDX_CHUNK)],
                              sem_e).wait()
    pltpu.sync_copy(rows_e, out_emb_hbm.at[pl.ds(base, B_PER_W)])
    cp_f.wait()


_sc_gather = pl.kernel(
    _sc_gather_body,
    out_type=(
        jax.ShapeDtypeStruct((BATCH, D_FEAT), jnp.float32),
        jax.ShapeDtypeStruct((BATCH, EMBED_SIZE), jnp.float32),
    ),
    mesh=plsc.VectorSubcoreMesh(core_axis_name="c", subcore_axis_name="s",
                                num_cores=NC, num_subcores=NS),
    compiler_params=pltpu.CompilerParams(use_tc_tiling_on_sc=False),
    scratch_types=[
        pltpu.VMEM((N_CHUNK, IDX_CHUNK), jnp.int32),
        pltpu.VMEM((N_CHUNK, IDX_CHUNK), jnp.int32),
        pltpu.VMEM((B_PER_W, D_FEAT), jnp.float32),
        pltpu.VMEM((B_PER_W, EMBED_SIZE), jnp.float32),
        pltpu.SemaphoreType.DMA,
        pltpu.SemaphoreType.DMA,
        pltpu.SemaphoreType.DMA,
    ],
)


def _proj_body(x_ref, w_ref, b_ref, o_ref):
    o_ref[...] = (jnp.dot(x_ref[...], w_ref[...],
                          preferred_element_type=jnp.float32) + b_ref[...])


_ROWS_PER_BLK = 2048


def _tc_proj(x, w_t, b2d):
    return pl.pallas_call(
        _proj_body,
        grid=(BATCH // _ROWS_PER_BLK,),
        in_specs=[
            pl.BlockSpec((_ROWS_PER_BLK, D_FEAT), lambda i: (i, 0)),
            pl.BlockSpec((D_FEAT, EMBED_SIZE), lambda i: (0, 0)),
            pl.BlockSpec((1, EMBED_SIZE), lambda i: (0, 0)),
        ],
        out_specs=pl.BlockSpec((_ROWS_PER_BLK, EMBED_SIZE), lambda i: (i, 0)),
        out_shape=jax.ShapeDtypeStruct((BATCH, EMBED_SIZE), jnp.float32),
    )(x, w_t, b2d)


def kernel(node_ids_feat, node_ids_embed, feat_table, proj_W, proj_b,
           embed_table):
    ids_f = node_ids_feat.astype(jnp.int32).reshape(NW, N_CHUNK, IDX_CHUNK)
    ids_e = node_ids_embed.astype(jnp.int32).reshape(NW, N_CHUNK, IDX_CHUNK)
    gathered, emb_embed = _sc_gather(ids_f, ids_e, feat_table, embed_table)
    emb_feat = _tc_proj(gathered, proj_W.T, proj_b.reshape(1, EMBED_SIZE))
    return (emb_feat, emb_embed)


# speedup vs baseline: 1.0221x; 1.0221x over previous
import functools

import jax
import jax.numpy as jnp
from jax import lax
from jax.experimental import pallas as pl
from jax.experimental.pallas import tpu as pltpu
from jax.experimental.pallas import tpu_sc as plsc

BATCH = 16384
D_FEAT = 128
EMBED_SIZE = 64

NC = 2
NS = 16
NW = NC * NS
B_PER_W = BATCH // NW
IDX_CHUNK = 128
N_CHUNK = B_PER_W // IDX_CHUNK

_SC_MESH = plsc.VectorSubcoreMesh(core_axis_name="c", subcore_axis_name="s",
                                  num_cores=NC, num_subcores=NS)


def _make_row_gather(width):
    def body(ids_hbm, tab_hbm, out_hbm, idx_v, rows_v, sem):
        wid = lax.axis_index("s") * NC + lax.axis_index("c")
        base = wid * B_PER_W
        pltpu.sync_copy(ids_hbm.at[wid], idx_v)
        for j in range(N_CHUNK):
            pltpu.async_copy(tab_hbm.at[idx_v.at[j]],
                             rows_v.at[pl.ds(j * IDX_CHUNK, IDX_CHUNK)], sem)
        for j in range(N_CHUNK):
            pltpu.make_async_copy(
                tab_hbm.at[idx_v.at[j]],
                rows_v.at[pl.ds(j * IDX_CHUNK, IDX_CHUNK)], sem).wait()
        pltpu.sync_copy(rows_v, out_hbm.at[pl.ds(base, B_PER_W)])

    return pl.kernel(
        body,
        out_type=jax.ShapeDtypeStruct((BATCH, width), jnp.float32),
        mesh=_SC_MESH,
        compiler_params=pltpu.CompilerParams(use_tc_tiling_on_sc=False),
        scratch_types=[
            pltpu.VMEM((N_CHUNK, IDX_CHUNK), jnp.int32),
            pltpu.VMEM((B_PER_W, width), jnp.float32),
            pltpu.SemaphoreType.DMA,
        ],
    )


_gather_feat = _make_row_gather(D_FEAT)
_gather_embed = _make_row_gather(EMBED_SIZE)


def _proj_body(x_ref, w_ref, b_ref, o_ref):
    o_ref[...] = (jnp.dot(w_ref[...], x_ref[...].T,
                          preferred_element_type=jnp.float32) + b_ref[...])


_ROWS_PER_BLK = 2048


def _tc_proj(x, w, b2d):
    return pl.pallas_call(
        _proj_body,
        grid=(BATCH // _ROWS_PER_BLK,),
        in_specs=[
            pl.BlockSpec((_ROWS_PER_BLK, D_FEAT), lambda i: (i, 0)),
            pl.BlockSpec((EMBED_SIZE, D_FEAT), lambda i: (0, 0)),
            pl.BlockSpec((EMBED_SIZE, 1), lambda i: (0, 0)),
        ],
        out_specs=pl.BlockSpec((EMBED_SIZE, _ROWS_PER_BLK), lambda i: (0, i)),
        out_shape=jax.ShapeDtypeStruct((EMBED_SIZE, BATCH), jnp.float32),
    )(x, w, b2d)


def kernel(node_ids_feat, node_ids_embed, feat_table, proj_W, proj_b,
           embed_table):
    ids_f = node_ids_feat.astype(jnp.int32).reshape(NW, N_CHUNK, IDX_CHUNK)
    ids_e = node_ids_embed.astype(jnp.int32).reshape(NW, N_CHUNK, IDX_CHUNK)
    emb_embed = _gather_embed(ids_e, embed_table)
    gathered = _gather_feat(ids_f, feat_table)
    feat_T = _tc_proj(gathered, proj_W, proj_b.reshape(EMBED_SIZE, 1))
    return (feat_T.T, emb_embed)
